# baseline (speedup 1.0000x reference)
import functools

import jax
import jax.numpy as jnp
from jax import lax
from jax.experimental import pallas as pl
from jax.experimental.pallas import tpu as pltpu
from jax.experimental.pallas import tpu_sc as plsc

D_MODEL = 64
NUM_CORES = 2
NUM_SUBCORES = 16
NW = NUM_CORES * NUM_SUBCORES
B = 4096
S = 200
CB = B // NW
L = 16


def _emb_kernel(table_hbm, idx_hbm, out_hbm,
                idx_v, hi_a, hi_b, buf_a, buf_b, out_a, out_b,
                gsem_a, gsem_b, wsem_a, wsem_b):
    wid = lax.axis_index("c") * NUM_SUBCORES + lax.axis_index("s")
    b0 = wid * CB
    pltpu.sync_copy(idx_hbm.at[wid], idx_v)

    lanes = lax.iota(jnp.int32, L)

    def prep_hi(s, hi_ref):
        for g in range(CB // L):
            xv = idx_v[s, pl.ds(g * L, L)]
            m = (xv >= HALF).astype(jnp.int32)
            hi_ref[pl.ds(g * L, L)] = xv - m * HALF

    def fire(s, hi_ref, buf, sem):
        prep_hi(s, hi_ref)
        pltpu.async_copy(table_hbm.at[hi_ref], buf, sem)

    def wait_gather(s, hi_ref, buf, sem):
        pltpu.make_async_copy(table_hbm.at[hi_ref], buf, sem).wait()

    def select(s, buf, out_t):
        for g in range(CB // L):
            xv = idx_v[s, pl.ds(g * L, L)]
            bids = lanes + (g * L)
            off = lax.shift_left((xv >= HALF).astype(jnp.int32), 6)

            @plsc.parallel_loop(0, D_MODEL, unroll=8)
            def _(d):
                out_t[d, pl.ds(g * L, L)] = plsc.load_gather(buf, [bids, off + d])

    def write(s, out_t, sem):
        pltpu.async_copy(out_t, out_hbm.at[s, :, pl.ds(b0, CB)], sem)

    def wait_write(s, out_t, sem):
        pltpu.make_async_copy(out_t, out_hbm.at[s, :, pl.ds(b0, CB)], sem).wait()

    fire(0, hi_a, buf_a, gsem_a)

    @pl.loop(0, S // 2)
    def _(g):
        s0 = g * 2
        s1 = s0 + 1
        fire(s1, hi_b, buf_b, gsem_b)
        wait_gather(s0, hi_a, buf_a, gsem_a)

        @pl.when(g > 0)
        def _():
            wait_write(s0 - 2, out_a, wsem_a)
        select(s0, buf_a, out_a)
        write(s0, out_a, wsem_a)

        @pl.when(g < S // 2 - 1)
        def _():
            fire(s0 + 2, hi_a, buf_a, gsem_a)
        wait_gather(s1, hi_b, buf_b, gsem_b)

        @pl.when(g > 0)
        def _():
            wait_write(s1 - 2, out_b, wsem_b)
        select(s1, buf_b, out_b)
        write(s1, out_b, wsem_b)

    wait_write(S - 2, out_a, wsem_a)
    wait_write(S - 1, out_b, wsem_b)


TBLK = 512
NTB = 977
HALF = NTB * TBLK


def _tpose_kernel(lo_ref, hi_ref, o_ref):
    o_ref[:, 0:64] = jnp.swapaxes(lo_ref[...], 0, 1)
    o_ref[:, 64:128] = jnp.swapaxes(hi_ref[...], 0, 1)


def _transpose_table(t_t):
    return pl.pallas_call(
        _tpose_kernel,
        grid=(NTB,),
        in_specs=[
            pl.BlockSpec((64, TBLK), lambda i: (0, i)),
            pl.BlockSpec((64, TBLK), lambda i: (0, i + NTB)),
        ],
        out_specs=pl.BlockSpec((TBLK, 128), lambda i: (i, 0)),
        out_shape=jax.ShapeDtypeStruct((HALF, 128), jnp.float32),
    )(t_t, t_t)


@jax.jit
def _embed(table128, idx3):
    run = functools.partial(
        pl.kernel,
        out_type=jax.ShapeDtypeStruct((S, D_MODEL, B), jnp.float32),
        mesh=plsc.VectorSubcoreMesh(core_axis_name="c", subcore_axis_name="s"),
        scratch_types=[
            pltpu.VMEM((S, CB), jnp.int32),
            pltpu.VMEM((CB,), jnp.int32),
            pltpu.VMEM((CB,), jnp.int32),
            pltpu.VMEM((CB, 128), jnp.float32),
            pltpu.VMEM((CB, 128), jnp.float32),
            pltpu.VMEM((D_MODEL, CB), jnp.float32),
            pltpu.VMEM((D_MODEL, CB), jnp.float32),
            pltpu.SemaphoreType.DMA,
            pltpu.SemaphoreType.DMA,
            pltpu.SemaphoreType.DMA,
            pltpu.SemaphoreType.DMA,
        ],
        compiler_params=pltpu.CompilerParams(needs_layout_passes=False),
    )(_emb_kernel)
    return run(table128, idx3)


def kernel(x, table):
    table128 = _transpose_table(table.T)
    idx3 = x.T.reshape(S, NW, CB).transpose(1, 0, 2).astype(jnp.int32)
    out_phys = _embed(table128, idx3)
    return out_phys.transpose(2, 0, 1)


# speedup vs baseline: 1.4548x; 1.4548x over previous
import functools

import jax
import jax.numpy as jnp
from jax import lax
from jax.experimental import pallas as pl
from jax.experimental.pallas import tpu as pltpu
from jax.experimental.pallas import tpu_sc as plsc

D_MODEL = 64
NUM_CORES = 2
NUM_SUBCORES = 16
NW = NUM_CORES * NUM_SUBCORES
B = 4096
S = 200
CB = B // NW
L = 16


def _emb_kernel(table_hbm, idx_hbm, out_hbm,
                idx_v, hi_a, hi_b, buf_a, buf_b, out_a, out_b,
                gsem_a, gsem_b, wsem_a, wsem_b):
    wid = lax.axis_index("c") * NUM_SUBCORES + lax.axis_index("s")
    b0 = wid * CB
    pltpu.sync_copy(idx_hbm.at[wid], idx_v)

    lanes = lax.iota(jnp.int32, L)

    def prep_hi(s, hi_ref):
        for g in range(CB // L):
            xv = idx_v[s, pl.ds(g * L, L)]
            m = (xv >= HALF).astype(jnp.int32)
            hi_ref[pl.ds(g * L, L)] = xv - m * HALF

    def fire(s, hi_ref, buf, sem):
        prep_hi(s, hi_ref)
        pltpu.async_copy(table_hbm.at[hi_ref], buf, sem)

    def wait_gather(s, hi_ref, buf, sem):
        pltpu.make_async_copy(table_hbm.at[hi_ref], buf, sem).wait()

    def select(s, buf, out_t):
        for g in range(CB // L):
            xv = idx_v[s, pl.ds(g * L, L)]
            bids = lanes + (g * L)
            off = lax.shift_left((xv >= HALF).astype(jnp.int32), 6)

            @plsc.parallel_loop(0, D_MODEL, unroll=8)
            def _(d):
                out_t[d, pl.ds(g * L, L)] = plsc.load_gather(buf, [bids, off + d])

    def write(s, out_t, sem):
        pltpu.async_copy(out_t, out_hbm.at[s, :, pl.ds(b0, CB)], sem)

    def wait_write(s, out_t, sem):
        pltpu.make_async_copy(out_t, out_hbm.at[s, :, pl.ds(b0, CB)], sem).wait()

    fire(0, hi_a, buf_a, gsem_a)

    @pl.loop(0, S // 2)
    def _(g):
        s0 = g * 2
        s1 = s0 + 1
        fire(s1, hi_b, buf_b, gsem_b)
        wait_gather(s0, hi_a, buf_a, gsem_a)

        @pl.when(g > 0)
        def _():
            wait_write(s0 - 2, out_a, wsem_a)
        select(s0, buf_a, out_a)
        write(s0, out_a, wsem_a)

        @pl.when(g < S // 2 - 1)
        def _():
            fire(s0 + 2, hi_a, buf_a, gsem_a)
        wait_gather(s1, hi_b, buf_b, gsem_b)

        @pl.when(g > 0)
        def _():
            wait_write(s1 - 2, out_b, wsem_b)
        select(s1, buf_b, out_b)
        write(s1, out_b, wsem_b)

    wait_write(S - 2, out_a, wsem_a)
    wait_write(S - 1, out_b, wsem_b)


TBLK = 4096
NTB = 123
HALF = NTB * TBLK


def _tpose_kernel(lo_ref, hi_ref, o_ref):
    o_ref[:, 0:64] = jnp.swapaxes(lo_ref[...], 0, 1)
    o_ref[:, 64:128] = jnp.swapaxes(hi_ref[...], 0, 1)


def _transpose_table(t_t):
    return pl.pallas_call(
        _tpose_kernel,
        grid=(NTB,),
        in_specs=[
            pl.BlockSpec((64, TBLK), lambda i: (0, i)),
            pl.BlockSpec(
                (64, TBLK),
                lambda i: (0, jnp.minimum(i + NTB, (1_000_000 + TBLK - 1) // TBLK - 1)),
            ),
        ],
        out_specs=pl.BlockSpec((TBLK, 128), lambda i: (i, 0)),
        out_shape=jax.ShapeDtypeStruct((HALF, 128), jnp.float32),
    )(t_t, t_t)


@jax.jit
def _embed(table128, idx3):
    run = functools.partial(
        pl.kernel,
        out_type=jax.ShapeDtypeStruct((S, D_MODEL, B), jnp.float32),
        mesh=plsc.VectorSubcoreMesh(core_axis_name="c", subcore_axis_name="s"),
        scratch_types=[
            pltpu.VMEM((S, CB), jnp.int32),
            pltpu.VMEM((CB,), jnp.int32),
            pltpu.VMEM((CB,), jnp.int32),
            pltpu.VMEM((CB, 128), jnp.float32),
            pltpu.VMEM((CB, 128), jnp.float32),
            pltpu.VMEM((D_MODEL, CB), jnp.float32),
            pltpu.VMEM((D_MODEL, CB), jnp.float32),
            pltpu.SemaphoreType.DMA,
            pltpu.SemaphoreType.DMA,
            pltpu.SemaphoreType.DMA,
            pltpu.SemaphoreType.DMA,
        ],
        compiler_params=pltpu.CompilerParams(needs_layout_passes=False),
    )(_emb_kernel)
    return run(table128, idx3)


def kernel(x, table):
    table128 = _transpose_table(table.T)
    idx3 = x.T.reshape(S, NW, CB).transpose(1, 0, 2).astype(jnp.int32)
    out_phys = _embed(table128, idx3)
    return out_phys.transpose(2, 0, 1)
